# baseline (speedup 1.0000x reference)
import functools
import jax
import jax.numpy as jnp
from jax.experimental import pallas as pl
from jax.experimental.pallas import tpu as pltpu

N = 10000
E = 160000
EPS = 1e-5


def _h_kernel(combine_ref, wn_ref, bn_ref, h_ref):
    h_ref[...] = (jnp.dot(combine_ref[...], wn_ref[...],
                          preferred_element_type=jnp.float32)
                  + bn_ref[...])


def _e32_kernel(ea_ref, we_ref, be_ref, e32_ref):
    e32_ref[...] = (jnp.dot(ea_ref[...], we_ref[...],
                            preferred_element_type=jnp.float32)
                    + be_ref[...])


def _pre(combine_out, p, edge_attr):
    h = pl.pallas_call(
        _h_kernel,
        out_shape=jax.ShapeDtypeStruct((N, 256), jnp.float32),
    )(combine_out, p['lin_node_W'], p['lin_node_b'][None])
    ea = jnp.pad(edge_attr, ((0, 0), (0, 6)))
    wep = jnp.pad(p['lin_edge_W'], ((0, 6), (0, 0)))
    grid = 16
    blk = E // grid
    e32 = pl.pallas_call(
        _e32_kernel,
        grid=(grid,),
        in_specs=[pl.BlockSpec((blk, 32), lambda i: (i, 0)),
                  pl.BlockSpec((32, 32), lambda i: (0, 0)),
                  pl.BlockSpec((1, 32), lambda i: (0, 0))],
        out_specs=pl.BlockSpec((blk, 32), lambda i: (i, 0)),
        out_shape=jax.ShapeDtypeStruct((E, 32), jnp.float32),
    )(ea, wep, p['lin_edge_b'][None])
    return h, e32


def _kqv_kernel(x_ref, wk_ref, wq_ref, wv_ref, ws_ref, bk_ref, bq_ref,
                bv_ref, bs_ref, k_ref, q_ref, v_ref, s_ref):
    x = x_ref[...]
    k_ref[...] = jnp.dot(x, wk_ref[...], preferred_element_type=jnp.float32) + bk_ref[...]
    q_ref[...] = jnp.dot(x, wq_ref[...], preferred_element_type=jnp.float32) + bq_ref[...]
    v_ref[...] = jnp.dot(x, wv_ref[...], preferred_element_type=jnp.float32) + bv_ref[...]
    s_ref[...] = jnp.dot(x, ws_ref[...], preferred_element_type=jnp.float32) + bs_ref[...]


def _kqv(x, p, li):
    shp = jax.ShapeDtypeStruct((N, 256), jnp.float32)
    return pl.pallas_call(
        _kqv_kernel,
        out_shape=(shp, shp, shp, shp),
    )(x, p['Wk'][li], p['Wq'][li], p['Wv'][li], p['Ws'][li],
      p['bk'][li][None], p['bq'][li][None], p['bv'][li][None],
      (p['bias'][li])[None])


def _edge_lin_kernel(e32_ref, we_ref, be_ref, e_ref):
    e_ref[...] = (jnp.dot(e32_ref[...], we_ref[...],
                          preferred_element_type=jnp.float32) + be_ref[...])


def _edge_lin(e32, p, li):
    grid = 16
    blk = E // grid
    return pl.pallas_call(
        _edge_lin_kernel,
        grid=(grid,),
        in_specs=[pl.BlockSpec((blk, 32), lambda i: (i, 0)),
                  pl.BlockSpec((32, 256), lambda i: (0, 0)),
                  pl.BlockSpec((1, 256), lambda i: (0, 0))],
        out_specs=pl.BlockSpec((blk, 256), lambda i: (i, 0)),
        out_shape=jax.ShapeDtypeStruct((E, 256), jnp.float32),
    )(e32, p['We'][li], p['be'][li][None])


def _post_kernel(agg_ref, skip_ref, sc_ref, bi_ref, out_ref):
    h = agg_ref[...] + skip_ref[...]
    mu = h.mean(-1, keepdims=True)
    var = ((h - mu) ** 2).mean(-1, keepdims=True)
    out_ref[...] = (h - mu) / jnp.sqrt(var + EPS) * sc_ref[...] + bi_ref[...]


def _post(agg, skip, p, li):
    return pl.pallas_call(
        _post_kernel,
        out_shape=jax.ShapeDtypeStruct((N, 256), jnp.float32),
    )(agg, skip, p['ln_scale'][li][None], p['ln_bias'][li][None])


def _final_kernel(h0_ref, h1_ref, h2_ref, w0_ref, w1_ref, w2_ref, bh_ref,
                  aw1_ref, ab1_ref, aw2_ref, ab2_ref, fw_ref, fb_ref,
                  out_ref):
    f = (jnp.dot(h0_ref[...], w0_ref[...], preferred_element_type=jnp.float32)
         + jnp.dot(h1_ref[...], w1_ref[...], preferred_element_type=jnp.float32)
         + jnp.dot(h2_ref[...], w2_ref[...], preferred_element_type=jnp.float32)
         + bh_ref[...])
    t = jnp.tanh(jnp.dot(f, aw1_ref[...], preferred_element_type=jnp.float32)
                 + ab1_ref[...])
    s = jnp.dot(t, aw2_ref[...], preferred_element_type=jnp.float32) + ab2_ref[0, 0]
    s = s[:, 0]
    m = jnp.max(s)
    w = jnp.exp(s - m)
    w = w / jnp.sum(w)
    pooled = jnp.sum(f * w[:, None], axis=0)
    out_ref[...] = jax.nn.sigmoid(
        jnp.sum(pooled * fw_ref[:, 0]) + fb_ref[0, 0])[None, None]


def _final(h0, h1, h2, p):
    w = p['lin_hidden_W']
    out = pl.pallas_call(
        _final_kernel,
        out_shape=jax.ShapeDtypeStruct((1, 1), jnp.float32),
    )(h0, h1, h2, w[0:256], w[256:512], w[512:768],
      p['lin_hidden_b'][None], p['attn_W1'], p['attn_b1'][None],
      p['attn_W2'], p['attn_b2'][None], p['final_W'], p['final_b'][None])
    return out[0, 0]


def kernel(combine_out, edge_attr, params, edge_index):
    p = params
    src = edge_index[0]
    dst = edge_index[1]
    h, e32 = _pre(combine_out, p, edge_attr)
    hiddens = []
    cur = h
    for li in range(3):
        k, q, v, skip = _kqv(cur, p, li)
        e = _edge_lin(e32, p, li)
        eta = jax.nn.sigmoid(jnp.take(k, dst, axis=0)
                             + jnp.take(q, src, axis=0) + e)
        msg = eta * jnp.take(v, src, axis=0)
        agg = jax.ops.segment_sum(msg, dst, num_segments=N)
        cur = _post(agg, skip, p, li)
        hiddens.append(cur)
    return _final(hiddens[0], hiddens[1], hiddens[2], p)
